# Initial kernel scaffold; baseline (speedup 1.0000x reference)
import functools

import jax
import jax.numpy as jnp
from jax import lax
from jax.experimental import pallas as pl
from jax.experimental.pallas import tpu as pltpu
from jax.experimental.pallas import tpu_sc as plsc

_N = 10000
_E = 320000
_D = 128
_NC = 2
_NS = 16
_NW = _NC * _NS
_EPT = _E // _NW
_CHUNK = 80
_NCHUNK = _EPT // _CHUNK
_STRIPE = _N // _NS

_mesh = plsc.VectorSubcoreMesh(core_axis_name="c", subcore_axis_name="s")


def _deg_body(src_hbm, dst_hbm, ones_hbm, zeros_hbm, degs_out, degd_out,
              sidx_v, didx_v, ones_v, degs_sh, degd_sh):
    cid = lax.axis_index("c")
    sid = lax.axis_index("s")
    tile = cid * _NS + sid
    row0 = sid * _STRIPE
    pltpu.sync_copy(zeros_hbm, degs_sh.at[pl.ds(row0, _STRIPE)])
    pltpu.sync_copy(zeros_hbm, degd_sh.at[pl.ds(row0, _STRIPE)])
    pltpu.sync_copy(ones_hbm, ones_v)
    pltpu.sync_copy(src_hbm.at[tile], sidx_v)
    pltpu.sync_copy(dst_hbm.at[tile], didx_v)
    plsc.subcore_barrier()

    def body(i, carry):
        pltpu.sync_copy(ones_v, degs_sh.at[sidx_v.at[i]], add=True)
        pltpu.sync_copy(ones_v, degd_sh.at[didx_v.at[i]], add=True)
        return carry

    lax.fori_loop(0, _NCHUNK, body, 0)
    plsc.subcore_barrier()
    pltpu.sync_copy(degs_sh.at[pl.ds(row0, _STRIPE)],
                    degs_out.at[cid, pl.ds(row0, _STRIPE)])
    pltpu.sync_copy(degd_sh.at[pl.ds(row0, _STRIPE)],
                    degd_out.at[cid, pl.ds(row0, _STRIPE)])


_deg_call = pl.kernel(
    _deg_body,
    out_type=(jax.ShapeDtypeStruct((_NC, _N, 16), jnp.float32),
              jax.ShapeDtypeStruct((_NC, _N, 16), jnp.float32)),
    mesh=_mesh,
    scratch_types=[
        pltpu.VMEM((_NCHUNK, _CHUNK), jnp.int32),
        pltpu.VMEM((_NCHUNK, _CHUNK), jnp.int32),
        pltpu.VMEM((_CHUNK, 16), jnp.float32),
        pltpu.VMEM_SHARED((_N, 16), jnp.float32),
        pltpu.VMEM_SHARED((_N, 16), jnp.float32),
    ],
)


def _msg_body(h_hbm, src_hbm, dst_hbm, zeros_hbm, agg_out,
              sidx_v, didx_v, rows_v, sem, agg_sh):
    cid = lax.axis_index("c")
    sid = lax.axis_index("s")
    tile = cid * _NS + sid
    row0 = sid * _STRIPE
    pltpu.sync_copy(zeros_hbm, agg_sh.at[pl.ds(row0, _STRIPE)])
    pltpu.sync_copy(src_hbm.at[tile], sidx_v)
    pltpu.sync_copy(dst_hbm.at[tile], didx_v)
    plsc.subcore_barrier()

    def body(i, carry):
        pltpu.async_copy(h_hbm.at[sidx_v.at[i]], rows_v, sem).wait()
        pltpu.sync_copy(rows_v, agg_sh.at[didx_v.at[i]], add=True)
        return carry

    lax.fori_loop(0, _NCHUNK, body, 0)
    plsc.subcore_barrier()
    pltpu.sync_copy(agg_sh.at[pl.ds(row0, _STRIPE)],
                    agg_out.at[cid, pl.ds(row0, _STRIPE)])


_msg_call = pl.kernel(
    _msg_body,
    out_type=jax.ShapeDtypeStruct((_NC, _N, _D), jnp.float32),
    mesh=_mesh,
    scratch_types=[
        pltpu.VMEM((_NCHUNK, _CHUNK), jnp.int32),
        pltpu.VMEM((_NCHUNK, _CHUNK), jnp.int32),
        pltpu.VMEM((_CHUNK, _D), jnp.float32),
        pltpu.SemaphoreType.DMA,
        pltpu.VMEM_SHARED((_N, _D), jnp.float32),
    ],
)

_BLK = 2000


def _prep_body(degs_ref, degd_ref, x_ref, ns_ref, nd_ref, xs_ref):
    ns = lax.rsqrt(jnp.maximum(degs_ref[0] + degs_ref[1], 1.0))
    nd = lax.rsqrt(jnp.maximum(degd_ref[0] + degd_ref[1], 1.0))
    ns_ref[...] = ns
    nd_ref[...] = nd
    xs_ref[...] = x_ref[...] * ns[:, 0:1]


def _prep_call(degs, degd, x):
    return pl.pallas_call(
        _prep_body,
        grid=(_N // _BLK,),
        in_specs=[
            pl.BlockSpec((_NC, _BLK, 16), lambda i: (0, i, 0)),
            pl.BlockSpec((_NC, _BLK, 16), lambda i: (0, i, 0)),
            pl.BlockSpec((_BLK, _D), lambda i: (i, 0)),
        ],
        out_specs=[
            pl.BlockSpec((_BLK, 16), lambda i: (i, 0)),
            pl.BlockSpec((_BLK, 16), lambda i: (i, 0)),
            pl.BlockSpec((_BLK, _D), lambda i: (i, 0)),
        ],
        out_shape=[
            jax.ShapeDtypeStruct((_N, 16), jnp.float32),
            jax.ShapeDtypeStruct((_N, 16), jnp.float32),
            jax.ShapeDtypeStruct((_N, _D), jnp.float32),
        ],
    )(degs, degd, x)


def _layer_body(agg_ref, nd_ref, ns_ref, w_ref, b_ref, o_ref, *, last):
    a = (agg_ref[0] + agg_ref[1]) * nd_ref[:, 0:1]
    y = jnp.dot(a, w_ref[...], preferred_element_type=jnp.float32) + b_ref[...]
    y = jnp.where(y > 0.0, y, jnp.expm1(y))
    if not last:
        y = y * ns_ref[:, 0:1]
    o_ref[...] = y


def _layer_call(agg, nd, ns, W, b, last):
    return pl.pallas_call(
        functools.partial(_layer_body, last=last),
        grid=(_N // _BLK,),
        in_specs=[
            pl.BlockSpec((_NC, _BLK, _D), lambda i: (0, i, 0)),
            pl.BlockSpec((_BLK, 16), lambda i: (i, 0)),
            pl.BlockSpec((_BLK, 16), lambda i: (i, 0)),
            pl.BlockSpec((_D, _D), lambda i: (0, 0)),
            pl.BlockSpec((1, _D), lambda i: (0, 0)),
        ],
        out_specs=pl.BlockSpec((_BLK, _D), lambda i: (i, 0)),
        out_shape=jax.ShapeDtypeStruct((_N, _D), jnp.float32),
    )(agg, nd, ns, W, b)


@jax.jit
def kernel(x, edge_index, W0, b0, W1, b1, W2, b2):
    src = edge_index[0].reshape(_NW, _NCHUNK, _CHUNK)
    dst = edge_index[1].reshape(_NW, _NCHUNK, _CHUNK)
    ones16 = jnp.ones((_CHUNK, 16), jnp.float32)
    z16 = jnp.zeros((_STRIPE, 16), jnp.float32)
    z128 = jnp.zeros((_STRIPE, _D), jnp.float32)

    degs, degd = _deg_call(src, dst, ones16, z16)
    ns, nd, h = _prep_call(degs, degd, x)
    for i, (W, b) in enumerate(((W0, b0), (W1, b1), (W2, b2))):
        agg = _msg_call(h, src, dst, z128)
        h = _layer_call(agg, nd, ns, W, b.reshape(1, _D), last=(i == 2))
    return h


# speedup vs baseline: 3.9943x; 3.9943x over previous
import functools

import jax
import jax.numpy as jnp
from jax import lax
from jax.experimental import pallas as pl
from jax.experimental.pallas import tpu as pltpu
from jax.experimental.pallas import tpu_sc as plsc

_N = 10000
_E = 320000
_D = 128
_NC = 2
_NS = 16
_NW = _NC * _NS
_EPT = _E // _NW
_CHUNK = 80
_NCHUNK = _EPT // _CHUNK
_NPAD = 10240
_STRIPE = _NPAD // _NS

_mesh = plsc.VectorSubcoreMesh(core_axis_name="c", subcore_axis_name="s")


def _count_body(idx_hbm, ones_hbm, zeros_hbm, cnt_out,
                idx_v, ones_v, acc_sh):
    cid = lax.axis_index("c")
    sid = lax.axis_index("s")
    tile = cid * _NS + sid
    row0 = sid * _STRIPE
    pltpu.sync_copy(zeros_hbm, acc_sh.at[pl.ds(row0, _STRIPE)])
    pltpu.sync_copy(ones_hbm, ones_v)
    plsc.subcore_barrier()

    def body(i, carry):
        base = tile * _EPT + i * _CHUNK
        pltpu.sync_copy(idx_hbm.at[pl.ds(base, _CHUNK)], idx_v)
        pltpu.sync_copy(ones_v, acc_sh.at[idx_v], add=True)
        return carry

    lax.fori_loop(0, _NCHUNK, body, 0)
    plsc.subcore_barrier()
    pltpu.sync_copy(acc_sh.at[pl.ds(row0, _STRIPE)],
                    cnt_out.at[cid, pl.ds(row0, _STRIPE)])


_count_call = pl.kernel(
    _count_body,
    out_type=jax.ShapeDtypeStruct((_NC, _NPAD, _D), jnp.float32),
    mesh=_mesh,
    scratch_types=[
        pltpu.VMEM((_CHUNK,), jnp.int32),
        pltpu.VMEM((_CHUNK, _D), jnp.float32),
        pltpu.VMEM_SHARED((_NPAD, _D), jnp.float32),
    ],
)


def _msg_body(h_hbm, src_hbm, dst_hbm, zeros_hbm, agg_out,
              sidx_v, didx_v, rows_v, sem, agg_sh):
    cid = lax.axis_index("c")
    sid = lax.axis_index("s")
    tile = cid * _NS + sid
    row0 = sid * _STRIPE
    pltpu.sync_copy(zeros_hbm, agg_sh.at[pl.ds(row0, _STRIPE)])
    plsc.subcore_barrier()

    def body(i, carry):
        base = tile * _EPT + i * _CHUNK
        pltpu.sync_copy(src_hbm.at[pl.ds(base, _CHUNK)], sidx_v)
        pltpu.sync_copy(dst_hbm.at[pl.ds(base, _CHUNK)], didx_v)
        pltpu.async_copy(h_hbm.at[sidx_v], rows_v, sem).wait()
        pltpu.sync_copy(rows_v, agg_sh.at[didx_v], add=True)
        return carry

    lax.fori_loop(0, _NCHUNK, body, 0)
    plsc.subcore_barrier()
    pltpu.sync_copy(agg_sh.at[pl.ds(row0, _STRIPE)],
                    agg_out.at[cid, pl.ds(row0, _STRIPE)])


_msg_call = pl.kernel(
    _msg_body,
    out_type=jax.ShapeDtypeStruct((_NC, _NPAD, _D), jnp.float32),
    mesh=_mesh,
    scratch_types=[
        pltpu.VMEM((_CHUNK,), jnp.int32),
        pltpu.VMEM((_CHUNK,), jnp.int32),
        pltpu.VMEM((_CHUNK, _D), jnp.float32),
        pltpu.SemaphoreType.DMA,
        pltpu.VMEM_SHARED((_NPAD, _D), jnp.float32),
    ],
)

_BLK = 2000


def _prep_body(degs_ref, degd_ref, x_ref, ns_ref, nd_ref, xs_ref):
    ns = lax.rsqrt(jnp.maximum(degs_ref[0] + degs_ref[1], 1.0))
    nd = lax.rsqrt(jnp.maximum(degd_ref[0] + degd_ref[1], 1.0))
    ns_ref[...] = ns
    nd_ref[...] = nd
    xs_ref[...] = x_ref[...] * ns[:, 0:1]


def _prep_call(degs, degd, x):
    return pl.pallas_call(
        _prep_body,
        grid=(_N // _BLK,),
        in_specs=[
            pl.BlockSpec((_NC, _BLK, 16), lambda i: (0, i, 0)),
            pl.BlockSpec((_NC, _BLK, 16), lambda i: (0, i, 0)),
            pl.BlockSpec((_BLK, _D), lambda i: (i, 0)),
        ],
        out_specs=[
            pl.BlockSpec((_BLK, 16), lambda i: (i, 0)),
            pl.BlockSpec((_BLK, 16), lambda i: (i, 0)),
            pl.BlockSpec((_BLK, _D), lambda i: (i, 0)),
        ],
        out_shape=[
            jax.ShapeDtypeStruct((_N, 16), jnp.float32),
            jax.ShapeDtypeStruct((_N, 16), jnp.float32),
            jax.ShapeDtypeStruct((_N, _D), jnp.float32),
        ],
    )(degs, degd, x)


def _layer_body(agg_ref, nd_ref, ns_ref, w_ref, b_ref, o_ref, *, last):
    a = (agg_ref[0] + agg_ref[1]) * nd_ref[:, 0:1]
    y = jnp.dot(a, w_ref[...], preferred_element_type=jnp.float32) + b_ref[...]
    y = jnp.where(y > 0.0, y, jnp.exp(jnp.minimum(y, 0.0)) - 1.0)
    if not last:
        y = y * ns_ref[:, 0:1]
    o_ref[...] = y


def _layer_call(agg, nd, ns, W, b, last):
    return pl.pallas_call(
        functools.partial(_layer_body, last=last),
        grid=(_N // _BLK,),
        in_specs=[
            pl.BlockSpec((_NC, _BLK, _D), lambda i: (0, i, 0)),
            pl.BlockSpec((_BLK, 16), lambda i: (i, 0)),
            pl.BlockSpec((_BLK, 16), lambda i: (i, 0)),
            pl.BlockSpec((_D, _D), lambda i: (0, 0)),
            pl.BlockSpec((1, _D), lambda i: (0, 0)),
        ],
        out_specs=pl.BlockSpec((_BLK, _D), lambda i: (i, 0)),
        out_shape=jax.ShapeDtypeStruct((_N, _D), jnp.float32),
    )(agg, nd, ns, W, b)


@jax.jit
def kernel(x, edge_index, W0, b0, W1, b1, W2, b2):
    src = edge_index[0]
    dst = edge_index[1]
    ones = jnp.ones((_CHUNK, _D), jnp.float32)
    z128 = jnp.zeros((_STRIPE, _D), jnp.float32)

    cnt_s = _count_call(src, ones, z128)
    cnt_d = _count_call(dst, ones, z128)
    degs = cnt_s[:, :, :16]
    degd = cnt_d[:, :, :16]
    ns, nd, h = _prep_call(degs, degd, x)
    for i, (W, b) in enumerate(((W0, b0), (W1, b1), (W2, b2))):
        agg = _msg_call(h, src, dst, z128)
        h = _layer_call(agg, nd, ns, W, b.reshape(1, _D), last=(i == 2))
    return h


# speedup vs baseline: 7.3313x; 1.8354x over previous
import functools

import jax
import jax.numpy as jnp
from jax import lax
from jax.experimental import pallas as pl
from jax.experimental.pallas import tpu as pltpu
from jax.experimental.pallas import tpu_sc as plsc

_N = 10000
_E = 320000
_D = 128
_NC = 2
_NS = 16
_NW = _NC * _NS
_EPT = _E // _NW
_CHUNK = 80
_NCHUNK = _EPT // _CHUNK
_NPAD = 10240
_STRIPE = _NPAD // _NS

_mesh = plsc.VectorSubcoreMesh(core_axis_name="c", subcore_axis_name="s")


def _count_body(idx_hbm, ones_hbm, zeros_hbm, cnt_out,
                idx_v, ones_v, acc_sh):
    cid = lax.axis_index("c")
    sid = lax.axis_index("s")
    tile = cid * _NS + sid
    row0 = sid * _STRIPE
    pltpu.sync_copy(zeros_hbm, acc_sh.at[pl.ds(row0, _STRIPE)])
    pltpu.sync_copy(ones_hbm, ones_v)
    plsc.subcore_barrier()

    def body(i, carry):
        base = tile * _EPT + i * _CHUNK
        pltpu.sync_copy(idx_hbm.at[pl.ds(base, _CHUNK)], idx_v)
        pltpu.sync_copy(ones_v, acc_sh.at[idx_v], add=True)
        return carry

    lax.fori_loop(0, _NCHUNK, body, 0)
    plsc.subcore_barrier()
    pltpu.sync_copy(acc_sh.at[pl.ds(row0, _STRIPE)],
                    cnt_out.at[cid, pl.ds(row0, _STRIPE)])


_count_call = pl.kernel(
    _count_body,
    out_type=jax.ShapeDtypeStruct((_NC, _NPAD, _D), jnp.float32),
    mesh=_mesh,
    scratch_types=[
        pltpu.VMEM((_CHUNK,), jnp.int32),
        pltpu.VMEM((_CHUNK, _D), jnp.float32),
        pltpu.VMEM_SHARED((_NPAD, _D), jnp.float32),
    ],
)


def _msg_body(h_hbm, src_hbm, dst_hbm, zeros_hbm, agg_out,
              srcall_v, didx0, didx1, rows0, rows1,
              sg0, sg1, sd0, sd1, agg_sh):
    cid = lax.axis_index("c")
    sid = lax.axis_index("s")
    tile = cid * _NS + sid
    ebase = tile * _EPT
    row0 = sid * _STRIPE
    didx = (didx0, didx1)
    rows = (rows0, rows1)
    sg = (sg0, sg1)
    sd = (sd0, sd1)

    pltpu.sync_copy(zeros_hbm, agg_sh.at[pl.ds(row0, _STRIPE)])
    pltpu.sync_copy(src_hbm.at[pl.ds(ebase, _EPT)], srcall_v)
    plsc.subcore_barrier()

    def issue(c, b):
        off = pl.multiple_of(c * _CHUNK, _CHUNK)
        pltpu.async_copy(dst_hbm.at[pl.ds(ebase + off, _CHUNK)], didx[b], sd[b])
        pltpu.async_copy(h_hbm.at[srcall_v.at[pl.ds(off, _CHUNK)]], rows[b], sg[b])

    def consume(c, b):
        pltpu.make_async_copy(dst_hbm.at[pl.ds(0, _CHUNK)], didx[b], sd[b]).wait()
        pltpu.make_async_copy(h_hbm.at[pl.ds(0, _CHUNK)], rows[b], sg[b]).wait()
        pltpu.sync_copy(rows[b], agg_sh.at[didx[b]], add=True)

    issue(0, 0)
    issue(1, 1)

    def body(i, carry):
        for b in (0, 1):
            c = i * 2 + b
            consume(c, b)

            @pl.when(c + 2 < _NCHUNK)
            def _():
                issue(c + 2, b)
        return carry

    lax.fori_loop(0, (_NCHUNK - 1) // 2, body, 0)
    consume(_NCHUNK - 1, (_NCHUNK - 1) % 2)
    plsc.subcore_barrier()
    pltpu.sync_copy(agg_sh.at[pl.ds(row0, _STRIPE)],
                    agg_out.at[cid, pl.ds(row0, _STRIPE)])


_msg_call = pl.kernel(
    _msg_body,
    out_type=jax.ShapeDtypeStruct((_NC, _NPAD, _D), jnp.float32),
    mesh=_mesh,
    scratch_types=[
        pltpu.VMEM((_EPT,), jnp.int32),
        pltpu.VMEM((_CHUNK,), jnp.int32),
        pltpu.VMEM((_CHUNK,), jnp.int32),
        pltpu.VMEM((_CHUNK, _D), jnp.float32),
        pltpu.VMEM((_CHUNK, _D), jnp.float32),
        pltpu.SemaphoreType.DMA,
        pltpu.SemaphoreType.DMA,
        pltpu.SemaphoreType.DMA,
        pltpu.SemaphoreType.DMA,
        pltpu.VMEM_SHARED((_NPAD, _D), jnp.float32),
    ],
)

_BLK = 2000


def _prep_body(degs_ref, degd_ref, x_ref, ns_ref, nd_ref, xs_ref):
    ns = lax.rsqrt(jnp.maximum(degs_ref[0] + degs_ref[1], 1.0))
    nd = lax.rsqrt(jnp.maximum(degd_ref[0] + degd_ref[1], 1.0))
    ns_ref[...] = ns
    nd_ref[...] = nd
    xs_ref[...] = x_ref[...] * ns[:, 0:1]


def _prep_call(degs, degd, x):
    return pl.pallas_call(
        _prep_body,
        grid=(_N // _BLK,),
        in_specs=[
            pl.BlockSpec((_NC, _BLK, 16), lambda i: (0, i, 0)),
            pl.BlockSpec((_NC, _BLK, 16), lambda i: (0, i, 0)),
            pl.BlockSpec((_BLK, _D), lambda i: (i, 0)),
        ],
        out_specs=[
            pl.BlockSpec((_BLK, 16), lambda i: (i, 0)),
            pl.BlockSpec((_BLK, 16), lambda i: (i, 0)),
            pl.BlockSpec((_BLK, _D), lambda i: (i, 0)),
        ],
        out_shape=[
            jax.ShapeDtypeStruct((_N, 16), jnp.float32),
            jax.ShapeDtypeStruct((_N, 16), jnp.float32),
            jax.ShapeDtypeStruct((_N, _D), jnp.float32),
        ],
    )(degs, degd, x)


def _layer_body(agg_ref, nd_ref, ns_ref, w_ref, b_ref, o_ref, *, last):
    a = (agg_ref[0] + agg_ref[1]) * nd_ref[:, 0:1]
    y = jnp.dot(a, w_ref[...], preferred_element_type=jnp.float32) + b_ref[...]
    y = jnp.where(y > 0.0, y, jnp.exp(jnp.minimum(y, 0.0)) - 1.0)
    if not last:
        y = y * ns_ref[:, 0:1]
    o_ref[...] = y


def _layer_call(agg, nd, ns, W, b, last):
    return pl.pallas_call(
        functools.partial(_layer_body, last=last),
        grid=(_N // _BLK,),
        in_specs=[
            pl.BlockSpec((_NC, _BLK, _D), lambda i: (0, i, 0)),
            pl.BlockSpec((_BLK, 16), lambda i: (i, 0)),
            pl.BlockSpec((_BLK, 16), lambda i: (i, 0)),
            pl.BlockSpec((_D, _D), lambda i: (0, 0)),
            pl.BlockSpec((1, _D), lambda i: (0, 0)),
        ],
        out_specs=pl.BlockSpec((_BLK, _D), lambda i: (i, 0)),
        out_shape=jax.ShapeDtypeStruct((_N, _D), jnp.float32),
    )(agg, nd, ns, W, b)


@jax.jit
def kernel(x, edge_index, W0, b0, W1, b1, W2, b2):
    src = edge_index[0]
    dst = edge_index[1]
    ones = jnp.ones((_CHUNK, _D), jnp.float32)
    z128 = jnp.zeros((_STRIPE, _D), jnp.float32)

    cnt_s = _count_call(src, ones, z128)
    cnt_d = _count_call(dst, ones, z128)
    degs = cnt_s[:, :, :16]
    degd = cnt_d[:, :, :16]
    ns, nd, h = _prep_call(degs, degd, x)
    for i, (W, b) in enumerate(((W0, b0), (W1, b1), (W2, b2))):
        agg = _msg_call(h, src, dst, z128)
        h = _layer_call(agg, nd, ns, W, b.reshape(1, _D), last=(i == 2))
    return h


# speedup vs baseline: 10.7855x; 1.4712x over previous
import functools

import jax
import jax.numpy as jnp
from jax import lax
from jax.experimental import pallas as pl
from jax.experimental.pallas import tpu as pltpu
from jax.experimental.pallas import tpu_sc as plsc

_N = 10000
_E = 320000
_D = 128
_NC = 2
_NS = 16
_NW = _NC * _NS
_EPT = _E // _NW
_CHUNK = 80
_NCHUNK = _EPT // _CHUNK
_NPAD = 10240
_STRIPE = _NPAD // _NS

_mesh = plsc.VectorSubcoreMesh(core_axis_name="c", subcore_axis_name="s")


_DCH = 128
_NDCH = _EPT // _DCH
_DTAIL = _EPT - _NDCH * _DCH


def _deg_body(src_hbm, dst_hbm, z1_hbm, degs_out, degd_out,
              s0, s1, d0, d1, sbt, dbt, ones_v, onest_v,
              ss0, ss1, sd0, sd1, degs_sh, degd_sh):
    cid = lax.axis_index("c")
    sid = lax.axis_index("s")
    tile = cid * _NS + sid
    ebase = tile * _EPT
    row0 = sid * _STRIPE
    sb = (s0, s1)
    db = (d0, d1)
    ss = (ss0, ss1)
    sd = (sd0, sd1)

    ov = jnp.ones((16,), jnp.float32)
    for j in range(_DCH // 16):
        ones_v[pl.ds(j * 16, 16)] = ov
    onest_v[pl.ds(0, 16)] = ov

    pltpu.sync_copy(z1_hbm, degs_sh.at[pl.ds(row0, _STRIPE)])
    pltpu.sync_copy(z1_hbm, degd_sh.at[pl.ds(row0, _STRIPE)])
    plsc.subcore_barrier()

    def issue(c, b):
        off = pl.multiple_of(c * _DCH, _DCH)
        pltpu.async_copy(src_hbm.at[pl.ds(ebase + off, _DCH)], sb[b], ss[b])
        pltpu.async_copy(dst_hbm.at[pl.ds(ebase + off, _DCH)], db[b], sd[b])

    def consume(b):
        pltpu.make_async_copy(src_hbm.at[pl.ds(0, _DCH)], sb[b], ss[b]).wait()
        pltpu.make_async_copy(dst_hbm.at[pl.ds(0, _DCH)], db[b], sd[b]).wait()
        pltpu.sync_copy(ones_v, degs_sh.at[sb[b]], add=True)
        pltpu.sync_copy(ones_v, degd_sh.at[db[b]], add=True)

    issue(0, 0)
    issue(1, 1)

    def body(i, carry):
        for b in (0, 1):
            c = i * 2 + b
            consume(b)

            @pl.when(c + 2 < _NDCH)
            def _():
                issue(c + 2, b)
        return carry

    lax.fori_loop(0, _NDCH // 2, body, 0)
    pltpu.sync_copy(src_hbm.at[pl.ds(ebase + _NDCH * _DCH, _DTAIL)], sbt)
    pltpu.sync_copy(dst_hbm.at[pl.ds(ebase + _NDCH * _DCH, _DTAIL)], dbt)
    pltpu.sync_copy(onest_v, degs_sh.at[sbt], add=True)
    pltpu.sync_copy(onest_v, degd_sh.at[dbt], add=True)
    plsc.subcore_barrier()
    pltpu.sync_copy(degs_sh.at[pl.ds(row0, _STRIPE)],
                    degs_out.at[cid, pl.ds(row0, _STRIPE)])
    pltpu.sync_copy(degd_sh.at[pl.ds(row0, _STRIPE)],
                    degd_out.at[cid, pl.ds(row0, _STRIPE)])


_deg_call = pl.kernel(
    _deg_body,
    out_type=(jax.ShapeDtypeStruct((_NC, _NPAD), jnp.float32),
              jax.ShapeDtypeStruct((_NC, _NPAD), jnp.float32)),
    mesh=_mesh,
    scratch_types=[
        pltpu.VMEM((_DCH,), jnp.int32),
        pltpu.VMEM((_DCH,), jnp.int32),
        pltpu.VMEM((_DCH,), jnp.int32),
        pltpu.VMEM((_DCH,), jnp.int32),
        pltpu.VMEM((_DTAIL,), jnp.int32),
        pltpu.VMEM((_DTAIL,), jnp.int32),
        pltpu.VMEM((_DCH,), jnp.float32),
        pltpu.VMEM((_DTAIL,), jnp.float32),
        pltpu.SemaphoreType.DMA,
        pltpu.SemaphoreType.DMA,
        pltpu.SemaphoreType.DMA,
        pltpu.SemaphoreType.DMA,
        pltpu.VMEM_SHARED((_NPAD,), jnp.float32),
        pltpu.VMEM_SHARED((_NPAD,), jnp.float32),
    ],
)


def _msg_body(h_hbm, src_hbm, dst_hbm, zeros_hbm, agg_out,
              srcall_v, didx0, didx1, rows0, rows1,
              sg0, sg1, sd0, sd1, agg_sh):
    cid = lax.axis_index("c")
    sid = lax.axis_index("s")
    tile = cid * _NS + sid
    ebase = tile * _EPT
    row0 = sid * _STRIPE
    didx = (didx0, didx1)
    rows = (rows0, rows1)
    sg = (sg0, sg1)
    sd = (sd0, sd1)

    pltpu.sync_copy(zeros_hbm, agg_sh.at[pl.ds(row0, _STRIPE)])
    pltpu.sync_copy(src_hbm.at[pl.ds(ebase, _EPT)], srcall_v)
    plsc.subcore_barrier()

    def issue(c, b):
        off = pl.multiple_of(c * _CHUNK, _CHUNK)
        pltpu.async_copy(dst_hbm.at[pl.ds(ebase + off, _CHUNK)], didx[b], sd[b])
        pltpu.async_copy(h_hbm.at[srcall_v.at[pl.ds(off, _CHUNK)]], rows[b], sg[b])

    def consume(c, b):
        pltpu.make_async_copy(dst_hbm.at[pl.ds(0, _CHUNK)], didx[b], sd[b]).wait()
        pltpu.make_async_copy(h_hbm.at[pl.ds(0, _CHUNK)], rows[b], sg[b]).wait()
        pltpu.sync_copy(rows[b], agg_sh.at[didx[b]], add=True)

    issue(0, 0)
    issue(1, 1)

    def body(i, carry):
        for b in (0, 1):
            c = i * 2 + b
            consume(c, b)

            @pl.when(c + 2 < _NCHUNK)
            def _():
                issue(c + 2, b)
        return carry

    lax.fori_loop(0, (_NCHUNK - 1) // 2, body, 0)
    consume(_NCHUNK - 1, (_NCHUNK - 1) % 2)
    plsc.subcore_barrier()
    pltpu.sync_copy(agg_sh.at[pl.ds(row0, _STRIPE)],
                    agg_out.at[cid, pl.ds(row0, _STRIPE)])


_msg_call = pl.kernel(
    _msg_body,
    out_type=jax.ShapeDtypeStruct((_NC, _NPAD, _D), jnp.float32),
    mesh=_mesh,
    scratch_types=[
        pltpu.VMEM((_EPT,), jnp.int32),
        pltpu.VMEM((_CHUNK,), jnp.int32),
        pltpu.VMEM((_CHUNK,), jnp.int32),
        pltpu.VMEM((_CHUNK, _D), jnp.float32),
        pltpu.VMEM((_CHUNK, _D), jnp.float32),
        pltpu.SemaphoreType.DMA,
        pltpu.SemaphoreType.DMA,
        pltpu.SemaphoreType.DMA,
        pltpu.SemaphoreType.DMA,
        pltpu.VMEM_SHARED((_NPAD, _D), jnp.float32),
    ],
)

_BLK = 2048
_GRID = (_N + _BLK - 1) // _BLK


def _prep_body(degs_ref, degd_ref, x_ref, ns_ref, nd_ref, xs_ref):
    ns = lax.rsqrt(jnp.maximum(degs_ref[0] + degs_ref[1], 1.0))
    nd = lax.rsqrt(jnp.maximum(degd_ref[0] + degd_ref[1], 1.0))
    ns2 = jnp.reshape(ns, (_BLK, 1))
    ns_ref[...] = ns2
    nd_ref[...] = jnp.reshape(nd, (_BLK, 1))
    xs_ref[...] = x_ref[...] * ns2


def _prep_call(degs, degd, x):
    return pl.pallas_call(
        _prep_body,
        grid=(_GRID,),
        in_specs=[
            pl.BlockSpec((_NC, _BLK), lambda i: (0, i)),
            pl.BlockSpec((_NC, _BLK), lambda i: (0, i)),
            pl.BlockSpec((_BLK, _D), lambda i: (i, 0)),
        ],
        out_specs=[
            pl.BlockSpec((_BLK, 1), lambda i: (i, 0)),
            pl.BlockSpec((_BLK, 1), lambda i: (i, 0)),
            pl.BlockSpec((_BLK, _D), lambda i: (i, 0)),
        ],
        out_shape=[
            jax.ShapeDtypeStruct((_N, 1), jnp.float32),
            jax.ShapeDtypeStruct((_N, 1), jnp.float32),
            jax.ShapeDtypeStruct((_N, _D), jnp.float32),
        ],
    )(degs, degd, x)


def _layer_body(agg_ref, nd_ref, ns_ref, w_ref, b_ref, o_ref, *, last):
    a = (agg_ref[0] + agg_ref[1]) * nd_ref[...]
    y = jnp.dot(a, w_ref[...], preferred_element_type=jnp.float32) + b_ref[...]
    y = jnp.where(y > 0.0, y, jnp.exp(jnp.minimum(y, 0.0)) - 1.0)
    if not last:
        y = y * ns_ref[...]
    o_ref[...] = y


def _layer_call(agg, nd, ns, W, b, last):
    return pl.pallas_call(
        functools.partial(_layer_body, last=last),
        grid=(_GRID,),
        in_specs=[
            pl.BlockSpec((_NC, _BLK, _D), lambda i: (0, i, 0)),
            pl.BlockSpec((_BLK, 1), lambda i: (i, 0)),
            pl.BlockSpec((_BLK, 1), lambda i: (i, 0)),
            pl.BlockSpec((_D, _D), lambda i: (0, 0)),
            pl.BlockSpec((1, _D), lambda i: (0, 0)),
        ],
        out_specs=pl.BlockSpec((_BLK, _D), lambda i: (i, 0)),
        out_shape=jax.ShapeDtypeStruct((_N, _D), jnp.float32),
    )(agg, nd, ns, W, b)


@jax.jit
def kernel(x, edge_index, W0, b0, W1, b1, W2, b2):
    src = edge_index[0]
    dst = edge_index[1]
    z1 = jnp.zeros((_STRIPE,), jnp.float32)
    z128 = jnp.zeros((_STRIPE, _D), jnp.float32)

    degs, degd = _deg_call(src, dst, z1)
    ns, nd, h = _prep_call(degs, degd, x)
    for i, (W, b) in enumerate(((W0, b0), (W1, b1), (W2, b2))):
        agg = _msg_call(h, src, dst, z128)
        h = _layer_call(agg, nd, ns, W, b.reshape(1, _D), last=(i == 2))
    return h


# speedup vs baseline: 12.5851x; 1.1669x over previous
import functools

import jax
import jax.numpy as jnp
from jax import lax
from jax.experimental import pallas as pl
from jax.experimental.pallas import tpu as pltpu
from jax.experimental.pallas import tpu_sc as plsc

_N = 10000
_E = 320000
_D = 128
_NC = 2
_NS = 16
_NW = _NC * _NS
_EPT = _E // _NW
_CHUNK = 80
_NCHUNK = _EPT // _CHUNK
_NPAD = 10240
_STRIPE = _NPAD // _NS

_mesh = plsc.VectorSubcoreMesh(core_axis_name="c", subcore_axis_name="s")


_DCH = 128
_NDCH = _EPT // _DCH
_DTAIL = _EPT - _NDCH * _DCH


def _deg_body(src_hbm, dst_hbm, z1_hbm, degs_out, degd_out,
              s0, s1, d0, d1, sbt, dbt, ones_v, onest_v,
              ss0, ss1, sd0, sd1, degs_sh, degd_sh):
    cid = lax.axis_index("c")
    sid = lax.axis_index("s")
    tile = cid * _NS + sid
    ebase = tile * _EPT
    row0 = sid * _STRIPE
    sb = (s0, s1)
    db = (d0, d1)
    ss = (ss0, ss1)
    sd = (sd0, sd1)

    ov = jnp.ones((16,), jnp.float32)
    for j in range(_DCH // 16):
        ones_v[pl.ds(j * 16, 16)] = ov
    onest_v[pl.ds(0, 16)] = ov

    pltpu.sync_copy(z1_hbm, degs_sh.at[pl.ds(row0, _STRIPE)])
    pltpu.sync_copy(z1_hbm, degd_sh.at[pl.ds(row0, _STRIPE)])
    plsc.subcore_barrier()

    def issue(c, b):
        off = pl.multiple_of(c * _DCH, _DCH)
        pltpu.async_copy(src_hbm.at[pl.ds(ebase + off, _DCH)], sb[b], ss[b])
        pltpu.async_copy(dst_hbm.at[pl.ds(ebase + off, _DCH)], db[b], sd[b])

    def consume(b):
        pltpu.make_async_copy(src_hbm.at[pl.ds(0, _DCH)], sb[b], ss[b]).wait()
        pltpu.make_async_copy(dst_hbm.at[pl.ds(0, _DCH)], db[b], sd[b]).wait()
        pltpu.sync_copy(ones_v, degs_sh.at[sb[b]], add=True)
        pltpu.sync_copy(ones_v, degd_sh.at[db[b]], add=True)

    issue(0, 0)
    issue(1, 1)

    def body(i, carry):
        for b in (0, 1):
            c = i * 2 + b
            consume(b)

            @pl.when(c + 2 < _NDCH)
            def _():
                issue(c + 2, b)
        return carry

    lax.fori_loop(0, _NDCH // 2, body, 0)
    pltpu.sync_copy(src_hbm.at[pl.ds(ebase + _NDCH * _DCH, _DTAIL)], sbt)
    pltpu.sync_copy(dst_hbm.at[pl.ds(ebase + _NDCH * _DCH, _DTAIL)], dbt)
    pltpu.sync_copy(onest_v, degs_sh.at[sbt], add=True)
    pltpu.sync_copy(onest_v, degd_sh.at[dbt], add=True)
    plsc.subcore_barrier()
    pltpu.sync_copy(degs_sh.at[pl.ds(row0, _STRIPE)],
                    degs_out.at[cid, pl.ds(row0, _STRIPE)])
    pltpu.sync_copy(degd_sh.at[pl.ds(row0, _STRIPE)],
                    degd_out.at[cid, pl.ds(row0, _STRIPE)])


_deg_call = pl.kernel(
    _deg_body,
    out_type=(jax.ShapeDtypeStruct((_NC, _NPAD), jnp.float32),
              jax.ShapeDtypeStruct((_NC, _NPAD), jnp.float32)),
    mesh=_mesh,
    scratch_types=[
        pltpu.VMEM((_DCH,), jnp.int32),
        pltpu.VMEM((_DCH,), jnp.int32),
        pltpu.VMEM((_DCH,), jnp.int32),
        pltpu.VMEM((_DCH,), jnp.int32),
        pltpu.VMEM((_DTAIL,), jnp.int32),
        pltpu.VMEM((_DTAIL,), jnp.int32),
        pltpu.VMEM((_DCH,), jnp.float32),
        pltpu.VMEM((_DTAIL,), jnp.float32),
        pltpu.SemaphoreType.DMA,
        pltpu.SemaphoreType.DMA,
        pltpu.SemaphoreType.DMA,
        pltpu.SemaphoreType.DMA,
        pltpu.VMEM_SHARED((_NPAD,), jnp.float32),
        pltpu.VMEM_SHARED((_NPAD,), jnp.float32),
    ],
)


_NBUF = 3


def _msg_body(h_hbm, src_hbm, dst_hbm, zeros_hbm, agg_out,
              srcall_v, didx0, didx1, didx2, rows0, rows1, rows2,
              sg0, sg1, sg2, sd0, sd1, sd2, agg_sh):
    cid = lax.axis_index("c")
    sid = lax.axis_index("s")
    tile = cid * _NS + sid
    ebase = tile * _EPT
    row0 = sid * _STRIPE
    didx = (didx0, didx1, didx2)
    rows = (rows0, rows1, rows2)
    sg = (sg0, sg1, sg2)
    sd = (sd0, sd1, sd2)

    pltpu.sync_copy(zeros_hbm, agg_sh.at[pl.ds(row0, _STRIPE)])
    pltpu.sync_copy(src_hbm.at[pl.ds(ebase, _EPT)], srcall_v)
    plsc.subcore_barrier()

    def issue(c, b):
        off = pl.multiple_of(c * _CHUNK, _CHUNK)
        pltpu.async_copy(dst_hbm.at[pl.ds(ebase + off, _CHUNK)], didx[b], sd[b])
        pltpu.async_copy(h_hbm.at[srcall_v.at[pl.ds(off, _CHUNK)]], rows[b], sg[b])

    def consume(b):
        pltpu.make_async_copy(dst_hbm.at[pl.ds(0, _CHUNK)], didx[b], sd[b]).wait()
        pltpu.make_async_copy(h_hbm.at[pl.ds(0, _CHUNK)], rows[b], sg[b]).wait()
        pltpu.sync_copy(rows[b], agg_sh.at[didx[b]], add=True)

    for b in range(_NBUF):
        issue(b, b)

    def body(i, carry):
        for b in range(_NBUF):
            c = i * _NBUF + b
            consume(b)

            @pl.when(c + _NBUF < _NCHUNK)
            def _():
                issue(c + _NBUF, b)
        return carry

    lax.fori_loop(0, _NCHUNK // _NBUF, body, 0)
    for c in range(_NBUF * (_NCHUNK // _NBUF), _NCHUNK):
        consume(c % _NBUF)
    plsc.subcore_barrier()
    pltpu.sync_copy(agg_sh.at[pl.ds(row0, _STRIPE)],
                    agg_out.at[cid, pl.ds(row0, _STRIPE)])


_msg_call = pl.kernel(
    _msg_body,
    out_type=jax.ShapeDtypeStruct((_NC, _NPAD, _D), jnp.float32),
    mesh=_mesh,
    scratch_types=(
        [pltpu.VMEM((_EPT,), jnp.int32)]
        + [pltpu.VMEM((_CHUNK,), jnp.int32) for _ in range(_NBUF)]
        + [pltpu.VMEM((_CHUNK, _D), jnp.float32) for _ in range(_NBUF)]
        + [pltpu.SemaphoreType.DMA for _ in range(2 * _NBUF)]
        + [pltpu.VMEM_SHARED((_NPAD, _D), jnp.float32)]
    ),
)

_BLK = 2048
_GRID = (_N + _BLK - 1) // _BLK


def _prep_body(degs_ref, degd_ref, x_ref, ns_ref, nd_ref, xs_ref):
    ns = lax.rsqrt(jnp.maximum(degs_ref[0] + degs_ref[1], 1.0))
    nd = lax.rsqrt(jnp.maximum(degd_ref[0] + degd_ref[1], 1.0))
    ns2 = jnp.reshape(ns, (_BLK, 1))
    ns_ref[...] = ns2
    nd_ref[...] = jnp.reshape(nd, (_BLK, 1))
    xs_ref[...] = x_ref[...] * ns2


def _prep_call(degs, degd, x):
    return pl.pallas_call(
        _prep_body,
        grid=(_GRID,),
        in_specs=[
            pl.BlockSpec((_NC, _BLK), lambda i: (0, i)),
            pl.BlockSpec((_NC, _BLK), lambda i: (0, i)),
            pl.BlockSpec((_BLK, _D), lambda i: (i, 0)),
        ],
        out_specs=[
            pl.BlockSpec((_BLK, 1), lambda i: (i, 0)),
            pl.BlockSpec((_BLK, 1), lambda i: (i, 0)),
            pl.BlockSpec((_BLK, _D), lambda i: (i, 0)),
        ],
        out_shape=[
            jax.ShapeDtypeStruct((_N, 1), jnp.float32),
            jax.ShapeDtypeStruct((_N, 1), jnp.float32),
            jax.ShapeDtypeStruct((_N, _D), jnp.float32),
        ],
    )(degs, degd, x)


def _layer_body(agg_ref, nd_ref, ns_ref, w_ref, b_ref, o_ref, *, last):
    a = (agg_ref[0] + agg_ref[1]) * nd_ref[...]
    y = jnp.dot(a, w_ref[...], preferred_element_type=jnp.float32) + b_ref[...]
    y = jnp.where(y > 0.0, y, jnp.exp(jnp.minimum(y, 0.0)) - 1.0)
    if not last:
        y = y * ns_ref[...]
    o_ref[...] = y


def _layer_call(agg, nd, ns, W, b, last):
    return pl.pallas_call(
        functools.partial(_layer_body, last=last),
        grid=(_GRID,),
        in_specs=[
            pl.BlockSpec((_NC, _BLK, _D), lambda i: (0, i, 0)),
            pl.BlockSpec((_BLK, 1), lambda i: (i, 0)),
            pl.BlockSpec((_BLK, 1), lambda i: (i, 0)),
            pl.BlockSpec((_D, _D), lambda i: (0, 0)),
            pl.BlockSpec((1, _D), lambda i: (0, 0)),
        ],
        out_specs=pl.BlockSpec((_BLK, _D), lambda i: (i, 0)),
        out_shape=jax.ShapeDtypeStruct((_N, _D), jnp.float32),
    )(agg, nd, ns, W, b)


@jax.jit
def kernel(x, edge_index, W0, b0, W1, b1, W2, b2):
    src = edge_index[0]
    dst = edge_index[1]
    z1 = jnp.zeros((_STRIPE,), jnp.float32)
    z128 = jnp.zeros((_STRIPE, _D), jnp.float32)

    degs, degd = _deg_call(src, dst, z1)
    ns, nd, h = _prep_call(degs, degd, x)
    for i, (W, b) in enumerate(((W0, b0), (W1, b1), (W2, b2))):
        agg = _msg_call(h, src, dst, z128)
        h = _layer_call(agg, nd, ns, W, b.reshape(1, _D), last=(i == 2))
    return h


# speedup vs baseline: 12.9910x; 1.0323x over previous
import functools

import jax
import jax.numpy as jnp
from jax import lax
from jax.experimental import pallas as pl
from jax.experimental.pallas import tpu as pltpu
from jax.experimental.pallas import tpu_sc as plsc

_N = 10000
_E = 320000
_D = 128
_NC = 2
_NS = 16
_NW = _NC * _NS
_EPT = _E // _NW
_CHUNK = 80
_NCHUNK = _EPT // _CHUNK
_NPAD = 10240
_STRIPE = _NPAD // _NS

_mesh = plsc.VectorSubcoreMesh(core_axis_name="c", subcore_axis_name="s")


_DCH = 128
_NDCH = _EPT // _DCH
_DTAIL = _EPT - _NDCH * _DCH


def _deg_body(src_hbm, dst_hbm, z1_hbm, degs_out, degd_out,
              s0, s1, s2, d0, d1, d2, sbt, dbt, ones_v, onest_v,
              ss0, ss1, ss2, sd0, sd1, sd2, degs_sh, degd_sh):
    cid = lax.axis_index("c")
    sid = lax.axis_index("s")
    tile = cid * _NS + sid
    ebase = tile * _EPT
    row0 = sid * _STRIPE
    sb = (s0, s1, s2)
    db = (d0, d1, d2)
    ss = (ss0, ss1, ss2)
    sd = (sd0, sd1, sd2)

    ov = jnp.ones((16,), jnp.float32)
    for j in range(_DCH // 16):
        ones_v[pl.ds(j * 16, 16)] = ov
    onest_v[pl.ds(0, 16)] = ov

    pltpu.sync_copy(z1_hbm, degs_sh.at[pl.ds(row0, _STRIPE)])
    pltpu.sync_copy(z1_hbm, degd_sh.at[pl.ds(row0, _STRIPE)])
    plsc.subcore_barrier()

    def issue(c, b):
        off = pl.multiple_of(c * _DCH, _DCH)
        pltpu.async_copy(src_hbm.at[pl.ds(ebase + off, _DCH)], sb[b], ss[b])
        pltpu.async_copy(dst_hbm.at[pl.ds(ebase + off, _DCH)], db[b], sd[b])

    def consume(b):
        pltpu.make_async_copy(src_hbm.at[pl.ds(0, _DCH)], sb[b], ss[b]).wait()
        pltpu.make_async_copy(dst_hbm.at[pl.ds(0, _DCH)], db[b], sd[b]).wait()
        pltpu.sync_copy(ones_v, degs_sh.at[sb[b]], add=True)
        pltpu.sync_copy(ones_v, degd_sh.at[db[b]], add=True)

    issue(0, 0)
    issue(1, 1)
    issue(2, 2)

    def body(i, carry):
        for b in (0, 1, 2):
            c = i * 3 + b
            consume(b)

            @pl.when(c + 3 < _NDCH)
            def _():
                issue(c + 3, b)
        return carry

    lax.fori_loop(0, _NDCH // 3, body, 0)
    pltpu.sync_copy(src_hbm.at[pl.ds(ebase + _NDCH * _DCH, _DTAIL)], sbt)
    pltpu.sync_copy(dst_hbm.at[pl.ds(ebase + _NDCH * _DCH, _DTAIL)], dbt)
    pltpu.sync_copy(onest_v, degs_sh.at[sbt], add=True)
    pltpu.sync_copy(onest_v, degd_sh.at[dbt], add=True)
    plsc.subcore_barrier()
    pltpu.sync_copy(degs_sh.at[pl.ds(row0, _STRIPE)],
                    degs_out.at[cid, pl.ds(row0, _STRIPE)])
    pltpu.sync_copy(degd_sh.at[pl.ds(row0, _STRIPE)],
                    degd_out.at[cid, pl.ds(row0, _STRIPE)])


_deg_call = pl.kernel(
    _deg_body,
    out_type=(jax.ShapeDtypeStruct((_NC, _NPAD), jnp.float32),
              jax.ShapeDtypeStruct((_NC, _NPAD), jnp.float32)),
    mesh=_mesh,
    scratch_types=[
        pltpu.VMEM((_DCH,), jnp.int32),
        pltpu.VMEM((_DCH,), jnp.int32),
        pltpu.VMEM((_DCH,), jnp.int32),
        pltpu.VMEM((_DCH,), jnp.int32),
        pltpu.VMEM((_DCH,), jnp.int32),
        pltpu.VMEM((_DCH,), jnp.int32),
        pltpu.VMEM((_DTAIL,), jnp.int32),
        pltpu.VMEM((_DTAIL,), jnp.int32),
        pltpu.VMEM((_DCH,), jnp.float32),
        pltpu.VMEM((_DTAIL,), jnp.float32),
        pltpu.SemaphoreType.DMA,
        pltpu.SemaphoreType.DMA,
        pltpu.SemaphoreType.DMA,
        pltpu.SemaphoreType.DMA,
        pltpu.SemaphoreType.DMA,
        pltpu.SemaphoreType.DMA,
        pltpu.VMEM_SHARED((_NPAD,), jnp.float32),
        pltpu.VMEM_SHARED((_NPAD,), jnp.float32),
    ],
)


_NBUF = 3


def _msg_body(h_hbm, src_hbm, dst_hbm, zeros_hbm, agg_out,
              srcall_v, didx0, didx1, didx2, rows0, rows1, rows2,
              sg0, sg1, sg2, sd0, sd1, sd2, sz, sa, agg_sh):
    cid = lax.axis_index("c")
    sid = lax.axis_index("s")
    tile = cid * _NS + sid
    ebase = tile * _EPT
    row0 = sid * _STRIPE
    didx = (didx0, didx1, didx2)
    rows = (rows0, rows1, rows2)
    sg = (sg0, sg1, sg2)
    sd = (sd0, sd1, sd2)

    pltpu.async_copy(zeros_hbm, agg_sh.at[pl.ds(row0, _STRIPE)], sz)
    pltpu.async_copy(src_hbm.at[pl.ds(ebase, _EPT)], srcall_v, sa)

    def issue_didx(c, b):
        off = pl.multiple_of(c * _CHUNK, _CHUNK)
        pltpu.async_copy(dst_hbm.at[pl.ds(ebase + off, _CHUNK)], didx[b], sd[b])

    def issue_rows(c, b):
        off = pl.multiple_of(c * _CHUNK, _CHUNK)
        pltpu.async_copy(h_hbm.at[srcall_v.at[pl.ds(off, _CHUNK)]], rows[b], sg[b])

    def issue(c, b):
        off = pl.multiple_of(c * _CHUNK, _CHUNK)
        pltpu.async_copy(dst_hbm.at[pl.ds(ebase + off, _CHUNK)], didx[b], sd[b])
        pltpu.async_copy(h_hbm.at[srcall_v.at[pl.ds(off, _CHUNK)]], rows[b], sg[b])

    def consume(b):
        pltpu.make_async_copy(dst_hbm.at[pl.ds(0, _CHUNK)], didx[b], sd[b]).wait()
        pltpu.make_async_copy(h_hbm.at[pl.ds(0, _CHUNK)], rows[b], sg[b]).wait()
        pltpu.sync_copy(rows[b], agg_sh.at[didx[b]], add=True)

    for b in range(_NBUF):
        issue_didx(b, b)
    pltpu.make_async_copy(src_hbm.at[pl.ds(0, _EPT)], srcall_v, sa).wait()
    for b in range(_NBUF):
        issue_rows(b, b)
    pltpu.make_async_copy(zeros_hbm, agg_sh.at[pl.ds(row0, _STRIPE)], sz).wait()
    plsc.subcore_barrier()

    def body(i, carry):
        for b in range(_NBUF):
            c = i * _NBUF + b
            consume(b)

            @pl.when(c + _NBUF < _NCHUNK)
            def _():
                issue(c + _NBUF, b)
        return carry

    lax.fori_loop(0, _NCHUNK // _NBUF, body, 0)
    for c in range(_NBUF * (_NCHUNK // _NBUF), _NCHUNK):
        consume(c % _NBUF)
    plsc.subcore_barrier()
    pltpu.sync_copy(agg_sh.at[pl.ds(row0, _STRIPE)],
                    agg_out.at[cid, pl.ds(row0, _STRIPE)])


_msg_call = pl.kernel(
    _msg_body,
    out_type=jax.ShapeDtypeStruct((_NC, _NPAD, _D), jnp.float32),
    mesh=_mesh,
    scratch_types=(
        [pltpu.VMEM((_EPT,), jnp.int32)]
        + [pltpu.VMEM((_CHUNK,), jnp.int32) for _ in range(_NBUF)]
        + [pltpu.VMEM((_CHUNK, _D), jnp.float32) for _ in range(_NBUF)]
        + [pltpu.SemaphoreType.DMA for _ in range(2 * _NBUF + 2)]
        + [pltpu.VMEM_SHARED((_NPAD, _D), jnp.float32)]
    ),
)

_BLK = 2048
_GRID = (_N + _BLK - 1) // _BLK


def _prep_body(degs_ref, degd_ref, x_ref, ns_ref, nd_ref, xs_ref):
    ns = lax.rsqrt(jnp.maximum(degs_ref[0] + degs_ref[1], 1.0))
    nd = lax.rsqrt(jnp.maximum(degd_ref[0] + degd_ref[1], 1.0))
    ns2 = jnp.reshape(ns, (_BLK, 1))
    ns_ref[...] = ns2
    nd_ref[...] = jnp.reshape(nd, (_BLK, 1))
    xs_ref[...] = x_ref[...] * ns2


def _prep_call(degs, degd, x):
    return pl.pallas_call(
        _prep_body,
        grid=(_GRID,),
        in_specs=[
            pl.BlockSpec((_NC, _BLK), lambda i: (0, i)),
            pl.BlockSpec((_NC, _BLK), lambda i: (0, i)),
            pl.BlockSpec((_BLK, _D), lambda i: (i, 0)),
        ],
        out_specs=[
            pl.BlockSpec((_BLK, 1), lambda i: (i, 0)),
            pl.BlockSpec((_BLK, 1), lambda i: (i, 0)),
            pl.BlockSpec((_BLK, _D), lambda i: (i, 0)),
        ],
        out_shape=[
            jax.ShapeDtypeStruct((_N, 1), jnp.float32),
            jax.ShapeDtypeStruct((_N, 1), jnp.float32),
            jax.ShapeDtypeStruct((_N, _D), jnp.float32),
        ],
    )(degs, degd, x)


def _layer_body(agg_ref, nd_ref, ns_ref, w_ref, b_ref, o_ref, *, last):
    a = (agg_ref[0] + agg_ref[1]) * nd_ref[...]
    y = jnp.dot(a, w_ref[...], preferred_element_type=jnp.float32) + b_ref[...]
    y = jnp.where(y > 0.0, y, jnp.exp(jnp.minimum(y, 0.0)) - 1.0)
    if not last:
        y = y * ns_ref[...]
    o_ref[...] = y


def _layer_call(agg, nd, ns, W, b, last):
    return pl.pallas_call(
        functools.partial(_layer_body, last=last),
        grid=(_GRID,),
        in_specs=[
            pl.BlockSpec((_NC, _BLK, _D), lambda i: (0, i, 0)),
            pl.BlockSpec((_BLK, 1), lambda i: (i, 0)),
            pl.BlockSpec((_BLK, 1), lambda i: (i, 0)),
            pl.BlockSpec((_D, _D), lambda i: (0, 0)),
            pl.BlockSpec((1, _D), lambda i: (0, 0)),
        ],
        out_specs=pl.BlockSpec((_BLK, _D), lambda i: (i, 0)),
        out_shape=jax.ShapeDtypeStruct((_N, _D), jnp.float32),
    )(agg, nd, ns, W, b)


@jax.jit
def kernel(x, edge_index, W0, b0, W1, b1, W2, b2):
    src = edge_index[0]
    dst = edge_index[1]
    z1 = jnp.zeros((_STRIPE,), jnp.float32)
    z128 = jnp.zeros((_STRIPE, _D), jnp.float32)

    degs, degd = _deg_call(src, dst, z1)
    ns, nd, h = _prep_call(degs, degd, x)
    for i, (W, b) in enumerate(((W0, b0), (W1, b1), (W2, b2))):
        agg = _msg_call(h, src, dst, z128)
        h = _layer_call(agg, nd, ns, W, b.reshape(1, _D), last=(i == 2))
    return h


# speedup vs baseline: 12.9982x; 1.0006x over previous
import functools

import jax
import jax.numpy as jnp
from jax import lax
from jax.experimental import pallas as pl
from jax.experimental.pallas import tpu as pltpu
from jax.experimental.pallas import tpu_sc as plsc

_N = 10000
_E = 320000
_D = 128
_NC = 2
_NS = 16
_NW = _NC * _NS
_EPT = _E // _NW
_CHUNK = 80
_NCHUNK = _EPT // _CHUNK
_NPAD = 10240
_STRIPE = _NPAD // _NS

_mesh = plsc.VectorSubcoreMesh(core_axis_name="c", subcore_axis_name="s")


_DCH = 128
_NDCH = _EPT // _DCH
_DTAIL = _EPT - _NDCH * _DCH


def _deg_body(src_hbm, dst_hbm, ones_hbm, z1_hbm, degs_out, degd_out,
              s0, s1, s2, d0, d1, d2, sbt, dbt, ones_v, onest_v,
              ss0, ss1, ss2, sd0, sd1, sd2, degs_sh, degd_sh):
    cid = lax.axis_index("c")
    sid = lax.axis_index("s")
    tile = cid * _NS + sid
    ebase = tile * _EPT
    row0 = sid * _STRIPE
    sb = (s0, s1, s2)
    db = (d0, d1, d2)
    ss = (ss0, ss1, ss2)
    sd = (sd0, sd1, sd2)

    pltpu.sync_copy(ones_hbm, ones_v)
    pltpu.sync_copy(ones_hbm.at[pl.ds(0, _DTAIL)], onest_v)
    pltpu.sync_copy(z1_hbm, degs_sh.at[pl.ds(row0, _STRIPE)])
    pltpu.sync_copy(z1_hbm, degd_sh.at[pl.ds(row0, _STRIPE)])
    plsc.subcore_barrier()

    def issue(c, b):
        off = pl.multiple_of(c * _DCH, _DCH)
        pltpu.async_copy(src_hbm.at[pl.ds(ebase + off, _DCH)], sb[b], ss[b])
        pltpu.async_copy(dst_hbm.at[pl.ds(ebase + off, _DCH)], db[b], sd[b])

    def consume(b):
        pltpu.make_async_copy(src_hbm.at[pl.ds(0, _DCH)], sb[b], ss[b]).wait()
        pltpu.make_async_copy(dst_hbm.at[pl.ds(0, _DCH)], db[b], sd[b]).wait()
        pltpu.sync_copy(ones_v, degs_sh.at[sb[b]], add=True)
        pltpu.sync_copy(ones_v, degd_sh.at[db[b]], add=True)

    issue(0, 0)
    issue(1, 1)
    issue(2, 2)

    def body(i, carry):
        for b in (0, 1, 2):
            c = i * 3 + b
            consume(b)

            @pl.when(c + 3 < _NDCH)
            def _():
                issue(c + 3, b)
        return carry

    lax.fori_loop(0, _NDCH // 3, body, 0)
    pltpu.sync_copy(src_hbm.at[pl.ds(ebase + _NDCH * _DCH, _DTAIL)], sbt)
    pltpu.sync_copy(dst_hbm.at[pl.ds(ebase + _NDCH * _DCH, _DTAIL)], dbt)
    pltpu.sync_copy(onest_v, degs_sh.at[sbt], add=True)
    pltpu.sync_copy(onest_v, degd_sh.at[dbt], add=True)
    plsc.subcore_barrier()
    pltpu.sync_copy(degs_sh.at[pl.ds(row0, _STRIPE)],
                    degs_out.at[cid, pl.ds(row0, _STRIPE)])
    pltpu.sync_copy(degd_sh.at[pl.ds(row0, _STRIPE)],
                    degd_out.at[cid, pl.ds(row0, _STRIPE)])


_deg_call = pl.kernel(
    _deg_body,
    out_type=(jax.ShapeDtypeStruct((_NC, _NPAD), jnp.float32),
              jax.ShapeDtypeStruct((_NC, _NPAD), jnp.float32)),
    mesh=_mesh,
    scratch_types=[
        pltpu.VMEM((_DCH,), jnp.int32),
        pltpu.VMEM((_DCH,), jnp.int32),
        pltpu.VMEM((_DCH,), jnp.int32),
        pltpu.VMEM((_DCH,), jnp.int32),
        pltpu.VMEM((_DCH,), jnp.int32),
        pltpu.VMEM((_DCH,), jnp.int32),
        pltpu.VMEM((_DTAIL,), jnp.int32),
        pltpu.VMEM((_DTAIL,), jnp.int32),
        pltpu.VMEM((_DCH,), jnp.float32),
        pltpu.VMEM((_DTAIL,), jnp.float32),
        pltpu.SemaphoreType.DMA,
        pltpu.SemaphoreType.DMA,
        pltpu.SemaphoreType.DMA,
        pltpu.SemaphoreType.DMA,
        pltpu.SemaphoreType.DMA,
        pltpu.SemaphoreType.DMA,
        pltpu.VMEM_SHARED((_NPAD,), jnp.float32),
        pltpu.VMEM_SHARED((_NPAD,), jnp.float32),
    ],
)


_NBUF = 3


def _msg_body(h_hbm, src_hbm, dst_hbm, zeros_hbm, agg_out,
              srcall_v, didx0, didx1, didx2, rows0, rows1, rows2,
              sg0, sg1, sg2, sd0, sd1, sd2, sz, sa, agg_sh):
    cid = lax.axis_index("c")
    sid = lax.axis_index("s")
    tile = cid * _NS + sid
    ebase = tile * _EPT
    row0 = sid * _STRIPE
    didx = (didx0, didx1, didx2)
    rows = (rows0, rows1, rows2)
    sg = (sg0, sg1, sg2)
    sd = (sd0, sd1, sd2)

    pltpu.async_copy(zeros_hbm, agg_sh.at[pl.ds(row0, _STRIPE)], sz)
    pltpu.async_copy(src_hbm.at[pl.ds(ebase, _EPT)], srcall_v, sa)

    def issue_didx(c, b):
        off = pl.multiple_of(c * _CHUNK, _CHUNK)
        pltpu.async_copy(dst_hbm.at[pl.ds(ebase + off, _CHUNK)], didx[b], sd[b])

    def issue_rows(c, b):
        off = pl.multiple_of(c * _CHUNK, _CHUNK)
        pltpu.async_copy(h_hbm.at[srcall_v.at[pl.ds(off, _CHUNK)]], rows[b], sg[b])

    def issue(c, b):
        off = pl.multiple_of(c * _CHUNK, _CHUNK)
        pltpu.async_copy(dst_hbm.at[pl.ds(ebase + off, _CHUNK)], didx[b], sd[b])
        pltpu.async_copy(h_hbm.at[srcall_v.at[pl.ds(off, _CHUNK)]], rows[b], sg[b])

    def consume(b):
        pltpu.make_async_copy(dst_hbm.at[pl.ds(0, _CHUNK)], didx[b], sd[b]).wait()
        pltpu.make_async_copy(h_hbm.at[pl.ds(0, _CHUNK)], rows[b], sg[b]).wait()
        pltpu.sync_copy(rows[b], agg_sh.at[didx[b]], add=True)

    for b in range(_NBUF):
        issue_didx(b, b)
    pltpu.make_async_copy(src_hbm.at[pl.ds(0, _EPT)], srcall_v, sa).wait()
    for b in range(_NBUF):
        issue_rows(b, b)
    pltpu.make_async_copy(zeros_hbm, agg_sh.at[pl.ds(row0, _STRIPE)], sz).wait()
    plsc.subcore_barrier()

    def body(i, carry):
        for b in range(_NBUF):
            c = i * _NBUF + b
            consume(b)

            @pl.when(c + _NBUF < _NCHUNK)
            def _():
                issue(c + _NBUF, b)
        return carry

    lax.fori_loop(0, _NCHUNK // _NBUF, body, 0)
    for c in range(_NBUF * (_NCHUNK // _NBUF), _NCHUNK):
        consume(c % _NBUF)
    plsc.subcore_barrier()
    pltpu.sync_copy(agg_sh.at[pl.ds(row0, _STRIPE)],
                    agg_out.at[cid, pl.ds(row0, _STRIPE)])


_msg_call = pl.kernel(
    _msg_body,
    out_type=jax.ShapeDtypeStruct((_NC, _NPAD, _D), jnp.float32),
    mesh=_mesh,
    scratch_types=(
        [pltpu.VMEM((_EPT,), jnp.int32)]
        + [pltpu.VMEM((_CHUNK,), jnp.int32) for _ in range(_NBUF)]
        + [pltpu.VMEM((_CHUNK, _D), jnp.float32) for _ in range(_NBUF)]
        + [pltpu.SemaphoreType.DMA for _ in range(2 * _NBUF + 2)]
        + [pltpu.VMEM_SHARED((_NPAD, _D), jnp.float32)]
    ),
)

_BLK = 2048
_GRID = (_N + _BLK - 1) // _BLK


def _prep_body(degs_ref, degd_ref, x_ref, ns_ref, nd_ref, xs_ref):
    ns = lax.rsqrt(jnp.maximum(degs_ref[0] + degs_ref[1], 1.0))
    nd = lax.rsqrt(jnp.maximum(degd_ref[0] + degd_ref[1], 1.0))
    ns2 = jnp.reshape(ns, (_BLK, 1))
    ns_ref[...] = ns2
    nd_ref[...] = jnp.reshape(nd, (_BLK, 1))
    xs_ref[...] = x_ref[...] * ns2


def _prep_call(degs, degd, x):
    return pl.pallas_call(
        _prep_body,
        grid=(_GRID,),
        in_specs=[
            pl.BlockSpec((_NC, _BLK), lambda i: (0, i)),
            pl.BlockSpec((_NC, _BLK), lambda i: (0, i)),
            pl.BlockSpec((_BLK, _D), lambda i: (i, 0)),
        ],
        out_specs=[
            pl.BlockSpec((_BLK, 1), lambda i: (i, 0)),
            pl.BlockSpec((_BLK, 1), lambda i: (i, 0)),
            pl.BlockSpec((_BLK, _D), lambda i: (i, 0)),
        ],
        out_shape=[
            jax.ShapeDtypeStruct((_N, 1), jnp.float32),
            jax.ShapeDtypeStruct((_N, 1), jnp.float32),
            jax.ShapeDtypeStruct((_N, _D), jnp.float32),
        ],
    )(degs, degd, x)


def _layer_body(agg_ref, nd_ref, ns_ref, w_ref, b_ref, o_ref, *, last):
    a = (agg_ref[0] + agg_ref[1]) * nd_ref[...]
    y = jnp.dot(a, w_ref[...], preferred_element_type=jnp.float32) + b_ref[...]
    y = jnp.where(y > 0.0, y, jnp.exp(jnp.minimum(y, 0.0)) - 1.0)
    if not last:
        y = y * ns_ref[...]
    o_ref[...] = y


def _layer_call(agg, nd, ns, W, b, last):
    return pl.pallas_call(
        functools.partial(_layer_body, last=last),
        grid=(_GRID,),
        in_specs=[
            pl.BlockSpec((_NC, _BLK, _D), lambda i: (0, i, 0)),
            pl.BlockSpec((_BLK, 1), lambda i: (i, 0)),
            pl.BlockSpec((_BLK, 1), lambda i: (i, 0)),
            pl.BlockSpec((_D, _D), lambda i: (0, 0)),
            pl.BlockSpec((1, _D), lambda i: (0, 0)),
        ],
        out_specs=pl.BlockSpec((_BLK, _D), lambda i: (i, 0)),
        out_shape=jax.ShapeDtypeStruct((_N, _D), jnp.float32),
    )(agg, nd, ns, W, b)


@jax.jit
def kernel(x, edge_index, W0, b0, W1, b1, W2, b2):
    src = edge_index[0]
    dst = edge_index[1]
    z1 = jnp.zeros((_STRIPE,), jnp.float32)
    z128 = jnp.zeros((_STRIPE, _D), jnp.float32)

    ones1 = jnp.ones((_DCH,), jnp.float32)
    degs, degd = _deg_call(src, dst, ones1, z1)
    ns, nd, h = _prep_call(degs, degd, x)
    for i, (W, b) in enumerate(((W0, b0), (W1, b1), (W2, b2))):
        agg = _msg_call(h, src, dst, z128)
        h = _layer_call(agg, nd, ns, W, b.reshape(1, _D), last=(i == 2))
    return h
